# Initial kernel scaffold; baseline (speedup 1.0000x reference)
import jax
import jax.numpy as jnp
from jax.experimental import pallas as pl


def kernel(x, magnitude):
    raise NotImplementedError("write your pallas kernel here")


# speedup vs baseline: 17.7933x; 17.7933x over previous
import functools

import jax
import jax.numpy as jnp
from jax import lax
from jax.experimental import pallas as pl
from jax.experimental.pallas import tpu as pltpu
from jax.experimental.pallas import tpu_sc as plsc

L = 16
NBINS = 256
NCH = 3
HSIZE = NCH * NBINS
CHUNK = 12288


def _build(B, H, W, C, in_dtype):
    assert C == NCH
    per_img = H * W * C
    half_len = per_img // 2
    assert half_len % CHUNK == 0
    n_chunks = half_len // CHUNK
    total_px = H * W
    n = B * per_img

    mesh = plsc.VectorSubcoreMesh(core_axis_name="c", subcore_axis_name="s")

    @functools.partial(
        pl.kernel,
        out_type=jax.ShapeDtypeStruct((n,), jnp.float32),
        mesh=mesh,
        compiler_params=pltpu.CompilerParams(needs_layout_passes=False),
        scratch_types=[
            pltpu.VMEM((CHUNK,), jnp.float32),
            pltpu.VMEM((CHUNK,), jnp.float32),
            pltpu.VMEM((L * HSIZE,), jnp.int32),
            pltpu.VMEM((HSIZE,), jnp.float32),
            pltpu.VMEM((HSIZE,), jnp.int32),
            pltpu.VMEM_SHARED((16, HSIZE), jnp.int32),
        ],
    )
    def eq_kernel(x_hbm, o_hbm, in_v, out_v, hist_v, lut_v, part_v, shr):
        c = lax.axis_index("c")
        s = lax.axis_index("s")
        img = c * (B // 2) + s // 2
        half = s % 2
        base = img * per_img + half * half_len

        iota = lax.broadcasted_iota(jnp.int32, (L,), 0)
        bank = iota * HSIZE
        ones = jnp.full((L,), 1, jnp.int32)
        zero = jnp.full((L,), 0, jnp.int32)
        choff = [((iota + u * L) % 3) * NBINS for u in range(3)]
        boff = [o + bank for o in choff]

        def zbody(i, _):
            hist_v[pl.ds(i * L, L)] = zero
            return 0
        lax.fori_loop(0, (L * HSIZE) // L, zbody, 0)

        def hist_chunk(ci, _):
            pltpu.sync_copy(x_hbm.at[pl.ds(base + ci * CHUNK, CHUNK)], in_v)
            def inner(t, _):
                b = t * 48
                for u in range(3):
                    v = in_v[pl.ds(b + u * L, L)]
                    vi = jnp.clip(v, 0.0, 255.0).astype(jnp.int32)
                    plsc.addupdate_scatter(hist_v, [vi + boff[u]], ones)
                return 0
            lax.fori_loop(0, CHUNK // 48, inner, 0)
            return 0
        lax.fori_loop(0, n_chunks, hist_chunk, 0)

        def red(j, _):
            acc = hist_v[pl.ds(j * L, L)]
            for bk in range(1, L):
                acc = acc + hist_v[pl.ds(bk * HSIZE + j * L, L)]
            hist_v[pl.ds(j * L, L)] = acc
            return 0
        lax.fori_loop(0, HSIZE // L, red, 0)

        pltpu.sync_copy(hist_v.at[pl.ds(0, HSIZE)], shr.at[s])
        plsc.subcore_barrier()
        pltpu.sync_copy(shr.at[s + 1 - 2 * half], part_v)
        def comb(j, _):
            hist_v[pl.ds(j * L, L)] = (
                hist_v[pl.ds(j * L, L)] + part_v[pl.ds(j * L, L)])
            return 0
        lax.fori_loop(0, HSIZE // L, comb, 0)

        for c3 in range(NCH):
            hb = c3 * NBINS

            def lastnz(k, cur):
                h = hist_v[pl.ds(hb + k * L, L)]
                m = jnp.max(jnp.where(h > 0, iota + k * L, -1))
                return jnp.maximum(cur, m)
            last_nz = lax.fori_loop(0, NBINS // L, lastnz, jnp.int32(-1))

            def hl(k, cur):
                h = hist_v[pl.ds(hb + k * L, L)]
                return cur + jnp.sum(jnp.where(iota + k * L == last_nz, h, 0))
            hlast = lax.fori_loop(0, NBINS // L, hl, jnp.int32(0))

            step = (jnp.int32(total_px) - hlast) // 255
            den = jnp.maximum(step, 1)
            half_step = step // 2
            is0 = step == 0

            def lutb(k, run):
                h = hist_v[pl.ds(hb + k * L, L)]
                excl = plsc.cumsum(h) - h + run
                lv = jnp.clip((excl + half_step) // den, 0, 255)
                lv = jnp.where(is0, iota + k * L, lv)
                lut_v[pl.ds(hb + k * L, L)] = lv.astype(jnp.float32)
                return run + jnp.sum(h)
            lax.fori_loop(0, NBINS // L, lutb, jnp.int32(0))

        def app_chunk(ci, _):
            pltpu.sync_copy(x_hbm.at[pl.ds(base + ci * CHUNK, CHUNK)], in_v)
            def inner(t, _):
                b = t * 48
                for u in range(3):
                    v = in_v[pl.ds(b + u * L, L)]
                    vi = jnp.clip(v, 0.0, 255.0).astype(jnp.int32)
                    out_v[pl.ds(b + u * L, L)] = plsc.load_gather(
                        lut_v, [vi + choff[u]])
                return 0
            lax.fori_loop(0, CHUNK // 48, inner, 0)
            pltpu.sync_copy(out_v, o_hbm.at[pl.ds(base + ci * CHUNK, CHUNK)])
            return 0
        lax.fori_loop(0, n_chunks, app_chunk, 0)

    return eq_kernel


def kernel(x, magnitude):
    B, H, W, C = x.shape
    eq = _build(B, H, W, C, x.dtype)
    out = eq(x.reshape(-1).astype(jnp.float32))
    return out.reshape(B, H, W, C).astype(x.dtype)


# speedup vs baseline: 308.8257x; 17.3563x over previous
import functools

import jax
import jax.numpy as jnp
from jax import lax
from jax.experimental import pallas as pl
from jax.experimental.pallas import tpu as pltpu
from jax.experimental.pallas import tpu_sc as plsc

L = 16
NBINS = 256
ROWS = 32
PLANES_PER_ROUND = 8


def _build(n_planes, H, W):
    rounds = n_planes // (2 * PLANES_PER_ROUND)
    half_rows = H // 2
    n_chunks = half_rows // ROWS
    total_px = H * W
    vregs_per_row = W // L

    mesh = plsc.VectorSubcoreMesh(core_axis_name="c", subcore_axis_name="s")

    @functools.partial(
        pl.kernel,
        out_type=jax.ShapeDtypeStruct((n_planes, H, W), jnp.float32),
        mesh=mesh,
        compiler_params=pltpu.CompilerParams(needs_layout_passes=False),
        scratch_types=[
            pltpu.VMEM((ROWS, W), jnp.float32),
            pltpu.VMEM((ROWS, W), jnp.float32),
            pltpu.VMEM((L * NBINS,), jnp.int32),
            pltpu.VMEM((NBINS,), jnp.int32),
            pltpu.VMEM((NBINS,), jnp.float32),
            pltpu.VMEM_SHARED((16, NBINS), jnp.int32),
        ],
    )
    def eq_kernel(x_hbm, o_hbm, in_v, out_v, hist_v, part_v, lut_v, shr):
        c = lax.axis_index("c")
        s = lax.axis_index("s")
        half = s % 2
        row0 = half * half_rows

        iota = lax.broadcasted_iota(jnp.int32, (L,), 0)
        bank = iota * NBINS
        ones = jnp.full((L,), 1, jnp.int32)
        zero = jnp.full((L,), 0, jnp.int32)

        for rnd in range(rounds):
            plane = (c * rounds + rnd) * PLANES_PER_ROUND + s // 2

            def zbody(i, _):
                hist_v[pl.ds(i * L, L)] = zero
                return 0
            lax.fori_loop(0, (L * NBINS) // L, zbody, 0)

            def hist_chunk(ci, _):
                pltpu.sync_copy(
                    x_hbm.at[plane, pl.ds(row0 + ci * ROWS, ROWS)], in_v)
                def hrow(r, _):
                    for u in range(vregs_per_row):
                        v = in_v[r, pl.ds(u * L, L)]
                        vi = jnp.clip(v, 0.0, 255.0).astype(jnp.int32)
                        plsc.addupdate_scatter(hist_v, [vi + bank], ones)
                    return 0
                lax.fori_loop(0, ROWS, hrow, 0)
                return 0
            lax.fori_loop(0, n_chunks, hist_chunk, 0)

            def red(j, _):
                acc = hist_v[pl.ds(j * L, L)]
                for bk in range(1, L):
                    acc = acc + hist_v[pl.ds(bk * NBINS + j * L, L)]
                hist_v[pl.ds(j * L, L)] = acc
                return 0
            lax.fori_loop(0, NBINS // L, red, 0)

            pltpu.sync_copy(hist_v.at[pl.ds(0, NBINS)], shr.at[s])
            plsc.subcore_barrier()
            pltpu.sync_copy(shr.at[s + 1 - 2 * half], part_v)
            def comb(j, _):
                hist_v[pl.ds(j * L, L)] = (
                    hist_v[pl.ds(j * L, L)] + part_v[pl.ds(j * L, L)])
                return 0
            lax.fori_loop(0, NBINS // L, comb, 0)
            plsc.subcore_barrier()

            def lastnz(k, cur):
                h = hist_v[pl.ds(k * L, L)]
                m = jnp.max(jnp.where(h > 0, iota + k * L, -1))
                return jnp.maximum(cur, m)
            last_nz = lax.fori_loop(0, NBINS // L, lastnz, jnp.int32(-1))

            def hl(k, cur):
                h = hist_v[pl.ds(k * L, L)]
                return cur + jnp.sum(jnp.where(iota + k * L == last_nz, h, 0))
            hlast = lax.fori_loop(0, NBINS // L, hl, jnp.int32(0))

            step = (jnp.int32(total_px) - hlast) // 255
            den = jnp.maximum(step, 1)
            half_step = step // 2
            is0 = step == 0

            def lutb(k, run):
                h = hist_v[pl.ds(k * L, L)]
                excl = plsc.cumsum(h) - h + run
                lv = jnp.clip((excl + half_step) // den, 0, 255)
                lv = jnp.where(is0, iota + k * L, lv)
                lut_v[pl.ds(k * L, L)] = lv.astype(jnp.float32)
                return run + jnp.sum(h)
            lax.fori_loop(0, NBINS // L, lutb, jnp.int32(0))

            def app_chunk(ci, _):
                pltpu.sync_copy(
                    x_hbm.at[plane, pl.ds(row0 + ci * ROWS, ROWS)], in_v)
                def arow(r, _):
                    for u in range(vregs_per_row):
                        v = in_v[r, pl.ds(u * L, L)]
                        vi = jnp.clip(v, 0.0, 255.0).astype(jnp.int32)
                        out_v[r, pl.ds(u * L, L)] = plsc.load_gather(
                            lut_v, [vi])
                    return 0
                lax.fori_loop(0, ROWS, arow, 0)
                pltpu.sync_copy(
                    out_v, o_hbm.at[plane, pl.ds(row0 + ci * ROWS, ROWS)])
                return 0
            lax.fori_loop(0, n_chunks, app_chunk, 0)

    return eq_kernel


def kernel(x, magnitude):
    B, H, W, C = x.shape
    xp = jnp.transpose(x, (0, 3, 1, 2)).reshape(B * C, H, W)
    eq = _build(B * C, H, W)
    out = eq(xp.astype(jnp.float32))
    return out.reshape(B, C, H, W).transpose(0, 2, 3, 1).astype(x.dtype)
